# baseline (speedup 1.0000x reference)
import functools

import jax
import jax.numpy as jnp
from jax import lax
from jax.experimental import pallas as pl
from jax.experimental.pallas import tpu as pltpu
from jax.experimental.pallas import tpu_sc as plsc

VOC_SIZE = 1000000
EMBED_DIM = 64
BATCH = 16384
HIST = 20

NC = 2
NS = 16
NW = NC * NS

TOTAL_ROWS = BATCH * HIST
ROWS_PER_W = TOTAL_ROWS // NW
CHUNK = 128
CPW = ROWS_PER_W // CHUNK
NCHUNKS = TOTAL_ROWS // CHUNK


def _body(idx_hbm, table_hbm, out_hbm, idx_v, buf_v, gsem):
    c = lax.axis_index("c")
    s = lax.axis_index("s")
    wid = s * NC + c
    pltpu.sync_copy(idx_hbm.at[wid], idx_v)

    def chunk_body(j, carry):
        pltpu.async_copy(table_hbm.at[idx_v.at[j]], buf_v, gsem).wait()
        pltpu.sync_copy(buf_v, out_hbm.at[wid * CPW + j])
        return carry

    lax.fori_loop(0, CPW, chunk_body, 0)


@jax.jit
def _gather(idx, table):
    mesh = plsc.VectorSubcoreMesh(core_axis_name="c", subcore_axis_name="s")
    kfn = pl.kernel(
        _body,
        out_type=jax.ShapeDtypeStruct((NCHUNKS, CHUNK, EMBED_DIM), jnp.float32),
        mesh=mesh,
        scratch_types=[
            pltpu.VMEM((CPW, CHUNK), jnp.int32),
            pltpu.VMEM((CHUNK, EMBED_DIM), jnp.float32),
            pltpu.SemaphoreType.DMA,
        ],
        compiler_params=pltpu.CompilerParams(use_tc_tiling_on_sc=False),
    )
    return kfn(idx, table)


def kernel(inputs, embeddings):
    idx = inputs.astype(jnp.int32).reshape(NW, CPW, CHUNK)
    out = _gather(idx, embeddings)
    return out.reshape(BATCH, HIST, EMBED_DIM)


# speedup vs baseline: 1.0656x; 1.0656x over previous
import jax
import jax.numpy as jnp
from jax import lax
from jax.experimental import pallas as pl
from jax.experimental.pallas import tpu as pltpu
from jax.experimental.pallas import tpu_sc as plsc

VOC_SIZE = 1000000
EMBED_DIM = 64
BATCH = 16384
HIST = 20

NC = 2
NS = 16
NW = NC * NS

TOTAL_ROWS = BATCH * HIST
ROWS_PER_W = TOTAL_ROWS // NW
CHUNK = 128
CPW = ROWS_PER_W // CHUNK
NCHUNKS = TOTAL_ROWS // CHUNK

M = 10
D = 5
T = CPW // M


def _body(idx_hbm, table_hbm, out_hbm, idx_v, bufs, gsem, osem):
    c = lax.axis_index("c")
    s = lax.axis_index("s")
    wid = s * NC + c
    out_base = wid * CPW
    pltpu.sync_copy(idx_hbm.at[wid], idx_v)

    def start_gather(j, b):
        pltpu.async_copy(table_hbm.at[idx_v.at[j]], bufs.at[b], gsem.at[b])

    def wait_gather(j, b):
        pltpu.make_async_copy(table_hbm.at[idx_v.at[j]], bufs.at[b],
                              gsem.at[b]).wait()

    def start_out(j, b):
        pltpu.async_copy(bufs.at[b], out_hbm.at[out_base + j], osem.at[b])

    def wait_out(j, b):
        pltpu.make_async_copy(bufs.at[b], out_hbm.at[out_base + j],
                              osem.at[b]).wait()

    for b in range(D):
        start_gather(b, b)

    def block(t, carry):
        j0 = t * M
        for b in range(M):
            j = j0 + b
            wait_gather(j, b)
            start_out(j, b)
            jn = j + D
            bn = (b + D) % M
            if b >= D:
                wait_out(jn - M, bn)

                @pl.when(jn < CPW)
                def _():
                    start_gather(jn, bn)
            else:
                @pl.when(t >= 1)
                def _():
                    wait_out(jn - M, bn)
                start_gather(jn, bn)
        return carry

    lax.fori_loop(0, T, block, 0)

    for b in range(D):
        j = CPW - D + b
        wait_out(j, j % M)


@jax.jit
def _gather(idx, table):
    mesh = plsc.VectorSubcoreMesh(core_axis_name="c", subcore_axis_name="s")
    kfn = pl.kernel(
        _body,
        out_type=jax.ShapeDtypeStruct((NCHUNKS, CHUNK, EMBED_DIM), jnp.float32),
        mesh=mesh,
        scratch_types=[
            pltpu.VMEM((CPW, CHUNK), jnp.int32),
            pltpu.VMEM((M, CHUNK, EMBED_DIM), jnp.float32),
            pltpu.SemaphoreType.DMA((M,)),
            pltpu.SemaphoreType.DMA((M,)),
        ],
        compiler_params=pltpu.CompilerParams(use_tc_tiling_on_sc=False),
    )
    return kfn(idx, table)


def kernel(inputs, embeddings):
    idx = inputs.astype(jnp.int32).reshape(NW, CPW, CHUNK)
    out = _gather(idx, embeddings)
    return out.reshape(BATCH, HIST, EMBED_DIM)
